# baseline (speedup 1.0000x reference)
import jax
import jax.numpy as jnp
from jax import lax
from jax.experimental import pallas as pl
from jax.experimental.pallas import tpu as pltpu
from jax.experimental.pallas import tpu_sc as plsc

N_NODES = 10000
N_PAD = 10240
E = 160000
E_PAD = 163840
CHUNK = 128
N_CHUNKS = E_PAD // (16 * CHUNK)
TRASH = 10100

ROW_BLK = 1024
GRID = N_PAD // ROW_BLK


def _l1_body(a0, a1, d0, d1, x, wl, wr, b1, h0, h1, h2, h3):
    deg = d0[...][:, :1] + d1[...][:, :1]
    r = 1.0 / jnp.maximum(deg, 1.0)
    acc = jnp.dot(a0[...] * r, wl[0:128, :], preferred_element_type=jnp.float32)
    acc += jnp.dot(a1[...] * r, wl[128:256, :], preferred_element_type=jnp.float32)
    acc += jnp.dot(x[...], wr[...], preferred_element_type=jnp.float32)
    h = jnp.maximum(acc + b1[...], 0.0)
    h0[...] = h[:, 0:128]
    h1[...] = h[:, 128:256]
    h2[...] = h[:, 256:384]
    h3[...] = h[:, 384:512]


def _layer1_tc(a0, a1, deg0, deg1, xp, W1_l, W1_r, b1):
    blk = lambda c: pl.BlockSpec((ROW_BLK, c), lambda i: (i, 0))
    whole = lambda s: pl.BlockSpec(s, lambda i: (0, 0))
    hs = jax.ShapeDtypeStruct((N_PAD, 128), jnp.float32)
    return pl.pallas_call(
        _l1_body,
        grid=(GRID,),
        in_specs=[blk(128), blk(128), blk(128), blk(128), blk(256),
                  whole((256, 512)), whole((256, 512)), whole((1, 512))],
        out_specs=[blk(128)] * 4,
        out_shape=[hs, hs, hs, hs],
    )(a0, a1, deg0, deg1, xp, W1_l, W1_r, b1.reshape(1, 512))


def _l2_body(a0, a1, a2, a3, d0, d1, h0, h1, h2, h3, wl, wr, b2, wc, bc, out):
    deg = d0[...][:, :1] + d1[...][:, :1]
    r = 1.0 / jnp.maximum(deg, 1.0)
    m = jnp.concatenate([a0[...] * r, a1[...] * r, a2[...] * r, a3[...] * r],
                        axis=1)
    h = jnp.concatenate([h0[...], h1[...], h2[...], h3[...]], axis=1)
    acc = jnp.dot(m, wl[...], preferred_element_type=jnp.float32)
    acc += jnp.dot(h, wr[...], preferred_element_type=jnp.float32)
    z = jnp.maximum(acc + b2[...], 0.0)
    o = jnp.dot(z, wc[...], preferred_element_type=jnp.float32) + bc[...]
    out[...] = jax.nn.sigmoid(o)


def _layer2_tc(aggs, deg0, deg1, hsplit, W2_l, W2_r, b2, Wc, bc):
    blk = lambda c: pl.BlockSpec((ROW_BLK, c), lambda i: (i, 0))
    whole = lambda s: pl.BlockSpec(s, lambda i: (0, 0))
    wc_pad = jnp.zeros((512, 128), jnp.float32).at[:, :16].set(Wc)
    bc_pad = jnp.zeros((1, 128), jnp.float32).at[0, :16].set(bc)
    return pl.pallas_call(
        _l2_body,
        grid=(GRID,),
        in_specs=[blk(128)] * 4 + [blk(128), blk(128)] + [blk(128)] * 4 +
                 [whole((512, 512)), whole((512, 512)), whole((1, 512)),
                  whole((512, 128)), whole((1, 128))],
        out_specs=blk(128),
        out_shape=jax.ShapeDtypeStruct((N_PAD, 128), jnp.float32),
    )(*aggs, deg0, deg1, *hsplit, W2_l, W2_r, b2.reshape(1, 512),
      wc_pad, bc_pad)


_SC_MESH = plsc.VectorSubcoreMesh(core_axis_name="c", subcore_axis_name="s")
_RPS = N_PAD // 16
_DEG_SPLIT = N_CHUNKS // 2


def _edge_pass(xb_h, idx_s, rows0, rows1, idx0, idx1, acc,
               semr0, semr1, semi0, semi1):
    def body(k, carry):
        pltpu.sync_copy(idx_s.at[k], idx0)
        pltpu.async_copy(xb_h.at[idx0.at[0]], rows0, semr0)
        pltpu.make_async_copy(xb_h.at[idx0.at[0]], rows0, semr0).wait()
        pltpu.sync_copy(rows0, acc.at[idx0.at[1]], add=True)
        return carry

    lax.fori_loop(0, N_CHUNKS, body, 0, unroll=False)


def _sc_agg1_body(x0, x1, idx_h, zrows,
                  agg0_o, agg1_o,
                  rows0, rows1, idx0, idx1, acc,
                  semr0, semr1, semi0, semi1):
    c = lax.axis_index("c")
    s = lax.axis_index("s")
    base = s * _RPS
    idx_s = idx_h.at[s]

    pltpu.sync_copy(zrows, acc.at[pl.ds(base, _RPS)])
    plsc.subcore_barrier()

    @pl.when(c == 0)
    def _():
        _edge_pass(x0, idx_s, rows0, rows1, idx0, idx1, acc,
                   semr0, semr1, semi0, semi1)

    @pl.when(c == 1)
    def _():
        _edge_pass(x1, idx_s, rows0, rows1, idx0, idx1, acc,
                   semr0, semr1, semi0, semi1)

    plsc.subcore_barrier()

    @pl.when(c == 0)
    def _():
        pltpu.sync_copy(acc.at[pl.ds(base, _RPS)],
                        agg0_o.at[pl.ds(base, _RPS)])

    @pl.when(c == 1)
    def _():
        pltpu.sync_copy(acc.at[pl.ds(base, _RPS)],
                        agg1_o.at[pl.ds(base, _RPS)])


def _sc_agg1(x0, x1, idx3, zrows):
    f = pl.kernel(
        _sc_agg1_body,
        out_type=[jax.ShapeDtypeStruct((N_PAD, 128), jnp.float32),
                  jax.ShapeDtypeStruct((N_PAD, 128), jnp.float32)],
        mesh=_SC_MESH,
        scratch_types=[
            pltpu.VMEM((CHUNK, 128), jnp.float32),
            pltpu.VMEM((CHUNK, 128), jnp.float32),
            pltpu.VMEM((2, CHUNK), jnp.int32),
            pltpu.VMEM((2, CHUNK), jnp.int32),
            pltpu.VMEM_SHARED((N_PAD, 128), jnp.float32),
            pltpu.SemaphoreType.DMA,
            pltpu.SemaphoreType.DMA,
            pltpu.SemaphoreType.DMA,
            pltpu.SemaphoreType.DMA,
        ],
    )
    return f(x0, x1, idx3, zrows)


def _sc_deg_body(idx_h, zrows, ones_h, deg0_o, deg1_o,
                 idxb, onesb, dega):
    c = lax.axis_index("c")
    s = lax.axis_index("s")
    base = s * _RPS
    idx_s = idx_h.at[s]
    pltpu.sync_copy(ones_h, onesb)
    pltpu.sync_copy(zrows, dega.at[pl.ds(base, _RPS)])
    plsc.subcore_barrier()

    def chunk(k, carry):
        pltpu.sync_copy(idx_s.at[k], idxb)
        pltpu.sync_copy(onesb, dega.at[idxb.at[1]], add=True)
        return carry

    @pl.when(c == 0)
    def _():
        lax.fori_loop(0, _DEG_SPLIT, chunk, 0, unroll=False)

    @pl.when(c == 1)
    def _():
        lax.fori_loop(_DEG_SPLIT, N_CHUNKS, chunk, 0, unroll=False)

    plsc.subcore_barrier()

    @pl.when(c == 0)
    def _():
        pltpu.sync_copy(dega.at[pl.ds(base, _RPS)],
                        deg0_o.at[pl.ds(base, _RPS)])

    @pl.when(c == 1)
    def _():
        pltpu.sync_copy(dega.at[pl.ds(base, _RPS)],
                        deg1_o.at[pl.ds(base, _RPS)])


def _sc_deg(idx3, zrows, ones_h):
    ds128 = jax.ShapeDtypeStruct((N_PAD, 128), jnp.float32)
    f = pl.kernel(
        _sc_deg_body,
        out_type=[ds128, ds128],
        mesh=_SC_MESH,
        scratch_types=[
            pltpu.VMEM((2, CHUNK), jnp.int32),
            pltpu.VMEM((CHUNK, 128), jnp.float32),
            pltpu.VMEM_SHARED((N_PAD, 128), jnp.float32),
        ],
    )
    return f(idx3, zrows, ones_h)


def _sc_agg2_body(h0, h1, h2, h3, idx_h, zrows,
                  o0, o1, o2, o3,
                  rows0, rows1, idx0, idx1, acc,
                  semr0, semr1, semi0, semi1):
    c = lax.axis_index("c")
    s = lax.axis_index("s")
    base = s * _RPS
    idx_s = idx_h.at[s]

    def one_pass(hb_h, out_h):
        pltpu.sync_copy(zrows, acc.at[pl.ds(base, _RPS)])
        plsc.subcore_barrier()
        _edge_pass(hb_h, idx_s, rows0, rows1, idx0, idx1, acc,
                   semr0, semr1, semi0, semi1)
        plsc.subcore_barrier()
        pltpu.sync_copy(acc.at[pl.ds(base, _RPS)], out_h.at[pl.ds(base, _RPS)])
        plsc.subcore_barrier()

    @pl.when(c == 0)
    def _():
        one_pass(h0, o0)
        one_pass(h2, o2)

    @pl.when(c == 1)
    def _():
        one_pass(h1, o1)
        one_pass(h3, o3)


def _sc_agg2(hsplit, idx3, zrows):
    hs = jax.ShapeDtypeStruct((N_PAD, 128), jnp.float32)
    f = pl.kernel(
        _sc_agg2_body,
        out_type=[hs, hs, hs, hs],
        mesh=_SC_MESH,
        scratch_types=[
            pltpu.VMEM((CHUNK, 128), jnp.float32),
            pltpu.VMEM((CHUNK, 128), jnp.float32),
            pltpu.VMEM((2, CHUNK), jnp.int32),
            pltpu.VMEM((2, CHUNK), jnp.int32),
            pltpu.VMEM_SHARED((N_PAD, 128), jnp.float32),
            pltpu.SemaphoreType.DMA,
            pltpu.SemaphoreType.DMA,
            pltpu.SemaphoreType.DMA,
            pltpu.SemaphoreType.DMA,
        ],
    )
    return f(*hsplit, idx3, zrows)


def _prep_edges(edge_index):
    ei = edge_index.astype(jnp.int32)
    src = jnp.full((E_PAD,), 0, jnp.int32).at[:E].set(ei[0])
    dst = jnp.full((E_PAD,), TRASH, jnp.int32).at[:E].set(ei[1])
    src3 = src.reshape(16, N_CHUNKS, CHUNK)
    dst3 = dst.reshape(16, N_CHUNKS, CHUNK)
    idx3 = jnp.stack([src3, dst3], axis=2)
    return idx3, dst3


def kernel(x, edge_index, W1_l, W1_r, b1, W2_l, W2_r, b2, Wc, bc):
    idx3, dst3 = _prep_edges(edge_index)

    xp = jnp.pad(x, ((0, N_PAD - N_NODES), (0, 0)))
    x0, x1 = xp[:, :128], xp[:, 128:]
    zrows = jnp.zeros((_RPS, 128), jnp.float32)
    ones_h = jnp.ones((CHUNK, 128), jnp.float32)

    deg0, deg1 = _sc_deg(idx3, zrows, ones_h)
    a0, a1 = _sc_agg1(x0, x1, idx3, zrows)

    hsplit = _layer1_tc(a0, a1, deg0, deg1, xp, W1_l, W1_r, b1)

    aggs2 = _sc_agg2(hsplit, idx3, zrows)

    out = _layer2_tc(aggs2, deg0, deg1, hsplit, W2_l, W2_r, b2, Wc, bc)
    return out[:N_NODES, :16]


# speedup vs baseline: 1.7785x; 1.7785x over previous
import jax
import jax.numpy as jnp
from jax import lax
from jax.experimental import pallas as pl
from jax.experimental.pallas import tpu as pltpu
from jax.experimental.pallas import tpu_sc as plsc

N_NODES = 10000
N_PAD = 10240
E = 160000
E_PAD = 163840
CHUNK = 128
N_CHUNKS = E_PAD // (16 * CHUNK)

ROW_BLK = 1024
GRID = N_PAD // ROW_BLK


def _l1_body(a0, a1, d0, d1, x, wl, wr, b1, h0, h1, h2, h3):
    deg = d0[...][:, :1] + d1[...][:, :1]
    r = 1.0 / jnp.maximum(deg, 1.0)
    acc = jnp.dot(a0[...] * r, wl[0:128, :], preferred_element_type=jnp.float32)
    acc += jnp.dot(a1[...] * r, wl[128:256, :], preferred_element_type=jnp.float32)
    acc += jnp.dot(x[...], wr[...], preferred_element_type=jnp.float32)
    h = jnp.maximum(acc + b1[...], 0.0)
    h0[...] = h[:, 0:128]
    h1[...] = h[:, 128:256]
    h2[...] = h[:, 256:384]
    h3[...] = h[:, 384:512]


def _layer1_tc(a0, a1, deg0, deg1, xp, W1_l, W1_r, b1):
    blk = lambda c: pl.BlockSpec((ROW_BLK, c), lambda i: (i, 0))
    whole = lambda s: pl.BlockSpec(s, lambda i: (0, 0))
    hs = jax.ShapeDtypeStruct((N_PAD, 128), jnp.float32)
    return pl.pallas_call(
        _l1_body,
        grid=(GRID,),
        in_specs=[blk(128), blk(128), blk(128), blk(128), blk(256),
                  whole((256, 512)), whole((256, 512)), whole((1, 512))],
        out_specs=[blk(128)] * 4,
        out_shape=[hs, hs, hs, hs],
    )(a0, a1, deg0, deg1, xp, W1_l, W1_r, b1.reshape(1, 512))


def _l2_body(a0, a1, a2, a3, d0, d1, h0, h1, h2, h3, wl, wr, b2, wc, bc, out):
    deg = d0[...][:, :1] + d1[...][:, :1]
    r = 1.0 / jnp.maximum(deg, 1.0)
    m = jnp.concatenate([a0[...] * r, a1[...] * r, a2[...] * r, a3[...] * r],
                        axis=1)
    h = jnp.concatenate([h0[...], h1[...], h2[...], h3[...]], axis=1)
    acc = jnp.dot(m, wl[...], preferred_element_type=jnp.float32)
    acc += jnp.dot(h, wr[...], preferred_element_type=jnp.float32)
    z = jnp.maximum(acc + b2[...], 0.0)
    o = jnp.dot(z, wc[...], preferred_element_type=jnp.float32) + bc[...]
    out[...] = jax.nn.sigmoid(o)


def _layer2_tc(aggs, deg0, deg1, hsplit, W2_l, W2_r, b2, Wc, bc):
    blk = lambda c: pl.BlockSpec((ROW_BLK, c), lambda i: (i, 0))
    whole = lambda s: pl.BlockSpec(s, lambda i: (0, 0))
    wc_pad = jnp.zeros((512, 128), jnp.float32).at[:, :16].set(Wc)
    bc_pad = jnp.zeros((1, 128), jnp.float32).at[0, :16].set(bc)
    return pl.pallas_call(
        _l2_body,
        grid=(GRID,),
        in_specs=[blk(128)] * 4 + [blk(128), blk(128)] + [blk(128)] * 4 +
                 [whole((512, 512)), whole((512, 512)), whole((1, 512)),
                  whole((512, 128)), whole((1, 128))],
        out_specs=blk(128),
        out_shape=jax.ShapeDtypeStruct((N_PAD, 128), jnp.float32),
    )(*aggs, deg0, deg1, *hsplit, W2_l, W2_r, b2.reshape(1, 512),
      wc_pad, bc_pad)


_SC_MESH = plsc.VectorSubcoreMesh(core_axis_name="c", subcore_axis_name="s")
_RPS = N_PAD // 16
_DEG_SPLIT = N_CHUNKS // 2


def _edge_pass(xb_h, idx_s, rows0, rows1, idx0, idx1, acc,
               semr0, semr1, semi0, semi1):
    def body(k, carry):
        pltpu.sync_copy(idx_s.at[k], idx0)
        pltpu.async_copy(xb_h.at[idx0.at[0]], rows0, semr0)
        pltpu.make_async_copy(xb_h.at[idx0.at[0]], rows0, semr0).wait()
        pltpu.sync_copy(rows0, acc.at[idx0.at[1]], add=True)
        return carry

    lax.fori_loop(0, N_CHUNKS, body, 0, unroll=False)


def _sc_agg1_body(x0, x1, idx_h, zrows,
                  agg0_o, agg1_o,
                  rows0, rows1, idx0, idx1, acc,
                  semr0, semr1, semi0, semi1):
    c = lax.axis_index("c")
    s = lax.axis_index("s")
    base = s * _RPS
    idx_s = idx_h.at[s]

    pltpu.sync_copy(zrows, acc.at[pl.ds(base, _RPS)])
    plsc.subcore_barrier()

    @pl.when(c == 0)
    def _():
        _edge_pass(x0, idx_s, rows0, rows1, idx0, idx1, acc,
                   semr0, semr1, semi0, semi1)

    @pl.when(c == 1)
    def _():
        _edge_pass(x1, idx_s, rows0, rows1, idx0, idx1, acc,
                   semr0, semr1, semi0, semi1)

    plsc.subcore_barrier()

    @pl.when(c == 0)
    def _():
        pltpu.sync_copy(acc.at[pl.ds(base, _RPS)],
                        agg0_o.at[pl.ds(base, _RPS)])

    @pl.when(c == 1)
    def _():
        pltpu.sync_copy(acc.at[pl.ds(base, _RPS)],
                        agg1_o.at[pl.ds(base, _RPS)])


def _sc_agg1(x0, x1, idx3, zrows):
    f = pl.kernel(
        _sc_agg1_body,
        out_type=[jax.ShapeDtypeStruct((N_PAD, 128), jnp.float32),
                  jax.ShapeDtypeStruct((N_PAD, 128), jnp.float32)],
        mesh=_SC_MESH,
        scratch_types=[
            pltpu.VMEM((CHUNK, 128), jnp.float32),
            pltpu.VMEM((CHUNK, 128), jnp.float32),
            pltpu.VMEM((2, CHUNK), jnp.int32),
            pltpu.VMEM((2, CHUNK), jnp.int32),
            pltpu.VMEM_SHARED((N_PAD, 128), jnp.float32),
            pltpu.SemaphoreType.DMA,
            pltpu.SemaphoreType.DMA,
            pltpu.SemaphoreType.DMA,
            pltpu.SemaphoreType.DMA,
        ],
    )
    return f(x0, x1, idx3, zrows)


def _sc_deg_body(idx_h, zrows, ones_h, deg0_o, deg1_o,
                 idxb, onesb, dega):
    c = lax.axis_index("c")
    s = lax.axis_index("s")
    base = s * _RPS
    idx_s = idx_h.at[s]
    pltpu.sync_copy(ones_h, onesb)
    pltpu.sync_copy(zrows, dega.at[pl.ds(base, _RPS)])
    plsc.subcore_barrier()

    def chunk(k, carry):
        pltpu.sync_copy(idx_s.at[k], idxb)
        pltpu.sync_copy(onesb, dega.at[idxb.at[1]], add=True)
        return carry

    @pl.when(c == 0)
    def _():
        lax.fori_loop(0, _DEG_SPLIT, chunk, 0, unroll=False)

    @pl.when(c == 1)
    def _():
        lax.fori_loop(_DEG_SPLIT, N_CHUNKS, chunk, 0, unroll=False)

    plsc.subcore_barrier()

    @pl.when(c == 0)
    def _():
        pltpu.sync_copy(dega.at[pl.ds(base, _RPS)],
                        deg0_o.at[pl.ds(base, _RPS)])

    @pl.when(c == 1)
    def _():
        pltpu.sync_copy(dega.at[pl.ds(base, _RPS)],
                        deg1_o.at[pl.ds(base, _RPS)])


def _sc_deg(idx3, zrows, ones_h):
    ds128 = jax.ShapeDtypeStruct((N_PAD, 128), jnp.float32)
    f = pl.kernel(
        _sc_deg_body,
        out_type=[ds128, ds128],
        mesh=_SC_MESH,
        scratch_types=[
            pltpu.VMEM((2, CHUNK), jnp.int32),
            pltpu.VMEM((CHUNK, 128), jnp.float32),
            pltpu.VMEM_SHARED((N_PAD, 128), jnp.float32),
        ],
    )
    return f(idx3, zrows, ones_h)


def _sc_agg2_body(h0, h1, h2, h3, idx_h, zrows,
                  o0, o1, o2, o3,
                  rows0, rows1, idx0, idx1, acc,
                  semr0, semr1, semi0, semi1):
    c = lax.axis_index("c")
    s = lax.axis_index("s")
    base = s * _RPS
    idx_s = idx_h.at[s]

    def one_pass(hb_h, out_h):
        pltpu.sync_copy(zrows, acc.at[pl.ds(base, _RPS)])
        plsc.subcore_barrier()
        _edge_pass(hb_h, idx_s, rows0, rows1, idx0, idx1, acc,
                   semr0, semr1, semi0, semi1)
        plsc.subcore_barrier()
        pltpu.sync_copy(acc.at[pl.ds(base, _RPS)], out_h.at[pl.ds(base, _RPS)])
        plsc.subcore_barrier()

    @pl.when(c == 0)
    def _():
        one_pass(h0, o0)
        one_pass(h2, o2)

    @pl.when(c == 1)
    def _():
        one_pass(h1, o1)
        one_pass(h3, o3)


def _sc_agg2(hsplit, idx3, zrows):
    hs = jax.ShapeDtypeStruct((N_PAD, 128), jnp.float32)
    f = pl.kernel(
        _sc_agg2_body,
        out_type=[hs, hs, hs, hs],
        mesh=_SC_MESH,
        scratch_types=[
            pltpu.VMEM((CHUNK, 128), jnp.float32),
            pltpu.VMEM((CHUNK, 128), jnp.float32),
            pltpu.VMEM((2, CHUNK), jnp.int32),
            pltpu.VMEM((2, CHUNK), jnp.int32),
            pltpu.VMEM_SHARED((N_PAD, 128), jnp.float32),
            pltpu.SemaphoreType.DMA,
            pltpu.SemaphoreType.DMA,
            pltpu.SemaphoreType.DMA,
            pltpu.SemaphoreType.DMA,
        ],
    )
    return f(*hsplit, idx3, zrows)


def _prep_edges(edge_index):
    ei = edge_index.astype(jnp.int32)
    pad = jnp.arange(E_PAD - E, dtype=jnp.int32)
    src = jnp.concatenate([ei[0], pad % 128])
    dst = jnp.concatenate([ei[1], N_NODES + pad % (N_PAD - N_NODES)])
    src3 = src.reshape(16, N_CHUNKS, CHUNK)
    dst3 = dst.reshape(16, N_CHUNKS, CHUNK)
    idx3 = jnp.stack([src3, dst3], axis=2)
    return idx3, dst3


def kernel(x, edge_index, W1_l, W1_r, b1, W2_l, W2_r, b2, Wc, bc):
    idx3, dst3 = _prep_edges(edge_index)

    xp = jnp.pad(x, ((0, N_PAD - N_NODES), (0, 0)))
    x0, x1 = xp[:, :128], xp[:, 128:]
    zrows = jnp.zeros((_RPS, 128), jnp.float32)
    ones_h = jnp.ones((CHUNK, 128), jnp.float32)

    deg0, deg1 = _sc_deg(idx3, zrows, ones_h)
    a0, a1 = _sc_agg1(x0, x1, idx3, zrows)

    hsplit = _layer1_tc(a0, a1, deg0, deg1, xp, W1_l, W1_r, b1)

    aggs2 = _sc_agg2(hsplit, idx3, zrows)

    out = _layer2_tc(aggs2, deg0, deg1, hsplit, W2_l, W2_r, b2, Wc, bc)
    return out[:N_NODES, :16]


# speedup vs baseline: 2.6801x; 1.5069x over previous
import jax
import jax.numpy as jnp
from jax import lax
from jax.experimental import pallas as pl
from jax.experimental.pallas import tpu as pltpu
from jax.experimental.pallas import tpu_sc as plsc

N_NODES = 10000
N_PAD = 10240
E = 160000
E_PAD = 163840
CHUNK = 128
N_CHUNKS = E_PAD // (16 * CHUNK)

ROW_BLK = 1024
GRID = N_PAD // ROW_BLK


def _l1_body(a0, a1, d0, d1, x, wl, wr, b1, h0, h1, h2, h3):
    deg = d0[...][:, :1] + d1[...][:, :1]
    r = 1.0 / jnp.maximum(deg, 1.0)
    acc = jnp.dot(a0[...] * r, wl[0:128, :], preferred_element_type=jnp.float32)
    acc += jnp.dot(a1[...] * r, wl[128:256, :], preferred_element_type=jnp.float32)
    acc += jnp.dot(x[...], wr[...], preferred_element_type=jnp.float32)
    h = jnp.maximum(acc + b1[...], 0.0)
    h0[...] = h[:, 0:128]
    h1[...] = h[:, 128:256]
    h2[...] = h[:, 256:384]
    h3[...] = h[:, 384:512]


def _layer1_tc(a0, a1, deg0, deg1, xp, W1_l, W1_r, b1):
    blk = lambda c: pl.BlockSpec((ROW_BLK, c), lambda i: (i, 0))
    whole = lambda s: pl.BlockSpec(s, lambda i: (0, 0))
    hs = jax.ShapeDtypeStruct((N_PAD, 128), jnp.float32)
    return pl.pallas_call(
        _l1_body,
        grid=(GRID,),
        in_specs=[blk(128), blk(128), blk(128), blk(128), blk(256),
                  whole((256, 512)), whole((256, 512)), whole((1, 512))],
        out_specs=[blk(128)] * 4,
        out_shape=[hs, hs, hs, hs],
    )(a0, a1, deg0, deg1, xp, W1_l, W1_r, b1.reshape(1, 512))


def _l2_body(a0, a1, a2, a3, d0, d1, h0, h1, h2, h3, wl, wr, b2, wc, bc, out):
    deg = d0[...][:, :1] + d1[...][:, :1]
    r = 1.0 / jnp.maximum(deg, 1.0)
    m = jnp.concatenate([a0[...] * r, a1[...] * r, a2[...] * r, a3[...] * r],
                        axis=1)
    h = jnp.concatenate([h0[...], h1[...], h2[...], h3[...]], axis=1)
    acc = jnp.dot(m, wl[...], preferred_element_type=jnp.float32)
    acc += jnp.dot(h, wr[...], preferred_element_type=jnp.float32)
    z = jnp.maximum(acc + b2[...], 0.0)
    o = jnp.dot(z, wc[...], preferred_element_type=jnp.float32) + bc[...]
    out[...] = jax.nn.sigmoid(o)


def _layer2_tc(aggs, deg0, deg1, hsplit, W2_l, W2_r, b2, Wc, bc):
    blk = lambda c: pl.BlockSpec((ROW_BLK, c), lambda i: (i, 0))
    whole = lambda s: pl.BlockSpec(s, lambda i: (0, 0))
    wc_pad = jnp.zeros((512, 128), jnp.float32).at[:, :16].set(Wc)
    bc_pad = jnp.zeros((1, 128), jnp.float32).at[0, :16].set(bc)
    return pl.pallas_call(
        _l2_body,
        grid=(GRID,),
        in_specs=[blk(128)] * 4 + [blk(128), blk(128)] + [blk(128)] * 4 +
                 [whole((512, 512)), whole((512, 512)), whole((1, 512)),
                  whole((512, 128)), whole((1, 128))],
        out_specs=blk(128),
        out_shape=jax.ShapeDtypeStruct((N_PAD, 128), jnp.float32),
    )(*aggs, deg0, deg1, *hsplit, W2_l, W2_r, b2.reshape(1, 512),
      wc_pad, bc_pad)


_SC_MESH = plsc.VectorSubcoreMesh(core_axis_name="c", subcore_axis_name="s")
_RPS = N_PAD // 16
_DEG_SPLIT = N_CHUNKS // 2


def _edge_pass(xb_h, idx_s, rows0, rows1, idx0, idx1, acc,
               semr0, semr1, semi0, semi1):
    npairs = N_CHUNKS // 2

    pltpu.sync_copy(idx_s.at[0], idx0)
    pltpu.async_copy(xb_h.at[idx0.at[0]], rows0, semr0)

    def body(p, carry):
        k1 = 2 * p + 1
        pltpu.sync_copy(idx_s.at[k1], idx1)
        pltpu.async_copy(xb_h.at[idx1.at[0]], rows1, semr1)
        pltpu.make_async_copy(xb_h.at[idx0.at[0]], rows0, semr0).wait()
        pltpu.sync_copy(rows0, acc.at[idx0.at[1]], add=True)
        k2 = jnp.minimum(2 * p + 2, N_CHUNKS - 2)
        pltpu.sync_copy(idx_s.at[k2], idx0)
        pltpu.async_copy(xb_h.at[idx0.at[0]], rows0, semr0)
        pltpu.make_async_copy(xb_h.at[idx1.at[0]], rows1, semr1).wait()
        pltpu.sync_copy(rows1, acc.at[idx1.at[1]], add=True)
        return carry

    lax.fori_loop(0, npairs, body, 0, unroll=False)
    pltpu.make_async_copy(xb_h.at[idx0.at[0]], rows0, semr0).wait()


def _sc_agg1_body(x0, x1, idx_h, zrows,
                  agg0_o, agg1_o,
                  rows0, rows1, idx0, idx1, acc,
                  semr0, semr1, semi0, semi1):
    c = lax.axis_index("c")
    s = lax.axis_index("s")
    base = s * _RPS
    idx_s = idx_h.at[s]

    pltpu.sync_copy(zrows, acc.at[pl.ds(base, _RPS)])
    plsc.subcore_barrier()

    @pl.when(c == 0)
    def _():
        _edge_pass(x0, idx_s, rows0, rows1, idx0, idx1, acc,
                   semr0, semr1, semi0, semi1)

    @pl.when(c == 1)
    def _():
        _edge_pass(x1, idx_s, rows0, rows1, idx0, idx1, acc,
                   semr0, semr1, semi0, semi1)

    plsc.subcore_barrier()

    @pl.when(c == 0)
    def _():
        pltpu.sync_copy(acc.at[pl.ds(base, _RPS)],
                        agg0_o.at[pl.ds(base, _RPS)])

    @pl.when(c == 1)
    def _():
        pltpu.sync_copy(acc.at[pl.ds(base, _RPS)],
                        agg1_o.at[pl.ds(base, _RPS)])


def _sc_agg1(x0, x1, idx3, zrows):
    f = pl.kernel(
        _sc_agg1_body,
        out_type=[jax.ShapeDtypeStruct((N_PAD, 128), jnp.float32),
                  jax.ShapeDtypeStruct((N_PAD, 128), jnp.float32)],
        mesh=_SC_MESH,
        scratch_types=[
            pltpu.VMEM((CHUNK, 128), jnp.float32),
            pltpu.VMEM((CHUNK, 128), jnp.float32),
            pltpu.VMEM((2, CHUNK), jnp.int32),
            pltpu.VMEM((2, CHUNK), jnp.int32),
            pltpu.VMEM_SHARED((N_PAD, 128), jnp.float32),
            pltpu.SemaphoreType.DMA,
            pltpu.SemaphoreType.DMA,
            pltpu.SemaphoreType.DMA,
            pltpu.SemaphoreType.DMA,
        ],
    )
    return f(x0, x1, idx3, zrows)


def _sc_deg_body(idx_h, zrows, ones_h, deg0_o, deg1_o,
                 idxb, onesb, dega):
    c = lax.axis_index("c")
    s = lax.axis_index("s")
    base = s * _RPS
    idx_s = idx_h.at[s]
    pltpu.sync_copy(ones_h, onesb)
    pltpu.sync_copy(zrows, dega.at[pl.ds(base, _RPS)])
    plsc.subcore_barrier()

    def chunk(k, carry):
        pltpu.sync_copy(idx_s.at[k], idxb)
        pltpu.sync_copy(onesb, dega.at[idxb.at[1]], add=True)
        return carry

    @pl.when(c == 0)
    def _():
        lax.fori_loop(0, _DEG_SPLIT, chunk, 0, unroll=False)

    @pl.when(c == 1)
    def _():
        lax.fori_loop(_DEG_SPLIT, N_CHUNKS, chunk, 0, unroll=False)

    plsc.subcore_barrier()

    @pl.when(c == 0)
    def _():
        pltpu.sync_copy(dega.at[pl.ds(base, _RPS)],
                        deg0_o.at[pl.ds(base, _RPS)])

    @pl.when(c == 1)
    def _():
        pltpu.sync_copy(dega.at[pl.ds(base, _RPS)],
                        deg1_o.at[pl.ds(base, _RPS)])


def _sc_deg(idx3, zrows, ones_h):
    ds128 = jax.ShapeDtypeStruct((N_PAD, 128), jnp.float32)
    f = pl.kernel(
        _sc_deg_body,
        out_type=[ds128, ds128],
        mesh=_SC_MESH,
        scratch_types=[
            pltpu.VMEM((2, CHUNK), jnp.int32),
            pltpu.VMEM((CHUNK, 128), jnp.float32),
            pltpu.VMEM_SHARED((N_PAD, 128), jnp.float32),
        ],
    )
    return f(idx3, zrows, ones_h)


def _sc_agg2_body(h0, h1, h2, h3, idx_h, zrows,
                  o0, o1, o2, o3,
                  rows0, rows1, idx0, idx1, acc,
                  semr0, semr1, semi0, semi1):
    c = lax.axis_index("c")
    s = lax.axis_index("s")
    base = s * _RPS
    idx_s = idx_h.at[s]

    def one_pass(hb_h, out_h):
        pltpu.sync_copy(zrows, acc.at[pl.ds(base, _RPS)])
        plsc.subcore_barrier()
        _edge_pass(hb_h, idx_s, rows0, rows1, idx0, idx1, acc,
                   semr0, semr1, semi0, semi1)
        plsc.subcore_barrier()
        pltpu.sync_copy(acc.at[pl.ds(base, _RPS)], out_h.at[pl.ds(base, _RPS)])
        plsc.subcore_barrier()

    @pl.when(c == 0)
    def _():
        one_pass(h0, o0)
        one_pass(h2, o2)

    @pl.when(c == 1)
    def _():
        one_pass(h1, o1)
        one_pass(h3, o3)


def _sc_agg2(hsplit, idx3, zrows):
    hs = jax.ShapeDtypeStruct((N_PAD, 128), jnp.float32)
    f = pl.kernel(
        _sc_agg2_body,
        out_type=[hs, hs, hs, hs],
        mesh=_SC_MESH,
        scratch_types=[
            pltpu.VMEM((CHUNK, 128), jnp.float32),
            pltpu.VMEM((CHUNK, 128), jnp.float32),
            pltpu.VMEM((2, CHUNK), jnp.int32),
            pltpu.VMEM((2, CHUNK), jnp.int32),
            pltpu.VMEM_SHARED((N_PAD, 128), jnp.float32),
            pltpu.SemaphoreType.DMA,
            pltpu.SemaphoreType.DMA,
            pltpu.SemaphoreType.DMA,
            pltpu.SemaphoreType.DMA,
        ],
    )
    return f(*hsplit, idx3, zrows)


def _prep_edges(edge_index):
    ei = edge_index.astype(jnp.int32)
    pad = jnp.arange(E_PAD - E, dtype=jnp.int32)
    src = jnp.concatenate([ei[0], pad % 128])
    dst = jnp.concatenate([ei[1], N_NODES + pad % (N_PAD - N_NODES)])
    src3 = src.reshape(16, N_CHUNKS, CHUNK)
    dst3 = dst.reshape(16, N_CHUNKS, CHUNK)
    idx3 = jnp.stack([src3, dst3], axis=2)
    return idx3, dst3


def kernel(x, edge_index, W1_l, W1_r, b1, W2_l, W2_r, b2, Wc, bc):
    idx3, dst3 = _prep_edges(edge_index)

    xp = jnp.pad(x, ((0, N_PAD - N_NODES), (0, 0)))
    x0, x1 = xp[:, :128], xp[:, 128:]
    zrows = jnp.zeros((_RPS, 128), jnp.float32)
    ones_h = jnp.ones((CHUNK, 128), jnp.float32)

    deg0, deg1 = _sc_deg(idx3, zrows, ones_h)
    a0, a1 = _sc_agg1(x0, x1, idx3, zrows)

    hsplit = _layer1_tc(a0, a1, deg0, deg1, xp, W1_l, W1_r, b1)

    aggs2 = _sc_agg2(hsplit, idx3, zrows)

    out = _layer2_tc(aggs2, deg0, deg1, hsplit, W2_l, W2_r, b2, Wc, bc)
    return out[:N_NODES, :16]
